# Initial kernel scaffold; baseline (speedup 1.0000x reference)
import jax
import jax.numpy as jnp
from jax.experimental import pallas as pl


def kernel(x, edge_index, edge_norm, W_rgc, W_u, W_i, Q):
    raise NotImplementedError("write your pallas kernel here")


# speedup vs baseline: 1.7095x; 1.7095x over previous
import functools

import jax
import jax.numpy as jnp
from jax import lax
from jax.experimental import pallas as pl
from jax.experimental.pallas import tpu as pltpu
from jax.experimental.pallas import tpu_sc as plsc

N_USERS = 5000
N_NODES = 10000
D = 256
HALF = 128
H1 = 128
N_EDGES = 320000

NC = 2
NS = 16
LANES = 16

EDGES_PER_TILE = N_EDGES // NS
CHUNK = 80
NCHUNKS = EDGES_PER_TILE // CHUNK
ROWS_A = 632
ROWS_B = N_NODES - 15 * ROWS_A


def _sc_agg_body(xr_hbm, src2_hbm, dst_hbm, norm16_hbm, out_hbm,
                 agg_sh, src_v, dst_v, norm_v, rows_v, sem):
    c = lax.axis_index("c")
    s = lax.axis_index("s")

    def zero_row(r, _):
        for j in range(HALF // LANES):
            rows_v[r, pl.ds(j * LANES, LANES)] = jnp.zeros((LANES,), jnp.float32)
        return 0
    lax.fori_loop(0, CHUNK, zero_row, 0)
    r0 = pl.multiple_of(s * ROWS_A, 8)

    @pl.when(s < 15)
    def _():
        for k in range(7):
            pltpu.sync_copy(rows_v, agg_sh.at[pl.ds(r0 + k * CHUNK, CHUNK)])
        pltpu.sync_copy(rows_v.at[pl.ds(0, 72)], agg_sh.at[pl.ds(r0 + 560, 72)])

    @pl.when(s == 15)
    def _():
        for k in range(6):
            pltpu.sync_copy(rows_v, agg_sh.at[pl.ds(15 * ROWS_A + k * CHUNK, CHUNK)])
        pltpu.sync_copy(rows_v.at[pl.ds(0, 40)],
                        agg_sh.at[pl.ds(15 * ROWS_A + 480, 40)])
    plsc.subcore_barrier()

    tile_base = s * EDGES_PER_TILE

    def chunk_body(k, _):
        base = tile_base + k * CHUNK
        pltpu.sync_copy(src2_hbm.at[pl.ds(c * N_EDGES + base, CHUNK)], src_v)
        pltpu.sync_copy(dst_hbm.at[pl.ds(base, CHUNK)], dst_v)
        pltpu.sync_copy(norm16_hbm.at[pl.ds(base * LANES, CHUNK * LANES)], norm_v)
        pltpu.async_copy(xr_hbm.at[src_v], rows_v, sem).wait()

        def row_body(r, _):
            nv = norm_v[pl.ds(r * LANES, LANES)]
            for j in range(HALF // LANES):
                sl = pl.ds(j * LANES, LANES)
                rows_v[r, sl] = rows_v[r, sl] * nv
            return 0
        lax.fori_loop(0, CHUNK, row_body, 0)

        pltpu.sync_copy(rows_v, agg_sh.at[dst_v], add=True)
        return 0
    lax.fori_loop(0, NCHUNKS, chunk_body, 0)

    plsc.subcore_barrier()

    @pl.when(s < 15)
    def _():
        pltpu.sync_copy(agg_sh.at[pl.ds(r0, ROWS_A)], out_hbm.at[c, pl.ds(r0, ROWS_A)])

    @pl.when(s == 15)
    def _():
        pltpu.sync_copy(agg_sh.at[pl.ds(15 * ROWS_A, ROWS_B)],
                        out_hbm.at[c, pl.ds(15 * ROWS_A, ROWS_B)])


_sc_agg = functools.partial(
    pl.kernel,
    mesh=plsc.VectorSubcoreMesh(core_axis_name="c", subcore_axis_name="s"),
    out_type=jax.ShapeDtypeStruct((NC, N_NODES, HALF), jnp.float32),
    scratch_types=[
        pltpu.VMEM_SHARED((N_NODES, HALF), jnp.float32),
        pltpu.VMEM((CHUNK,), jnp.int32),
        pltpu.VMEM((CHUNK,), jnp.int32),
        pltpu.VMEM((CHUNK * LANES,), jnp.float32),
        pltpu.VMEM((CHUNK, HALF), jnp.float32),
        pltpu.SemaphoreType.DMA,
    ],
)(_sc_agg_body)


ROWB = 1000
NBLK_U = N_USERS // ROWB


def _enc_body(aL_ref, aR_ref, Wr_ref, Wu_ref, Wi_ref, Q_ref, out_ref):
    b = pl.program_id(0)
    aL = aL_ref[0]
    aR = aR_ref[0]
    Wr = Wr_ref[...]
    feats = jnp.maximum(
        jnp.dot(aL, Wr[:HALF], preferred_element_type=jnp.float32)
        + jnp.dot(aR, Wr[HALF:], preferred_element_type=jnp.float32), 0.0)
    is_user = b < NBLK_U
    W2 = jnp.where(is_user, Wu_ref[...], Wi_ref[...])
    h = jnp.maximum(jnp.dot(feats, W2, preferred_element_type=jnp.float32), 0.0)
    hq = jnp.dot(h, Q_ref[...], preferred_element_type=jnp.float32)
    out_ref[...] = jnp.where(is_user, hq, h)


def _encode(agg2, W_rgc, W_u, W_i, Q):
    wspec = lambda shape: pl.BlockSpec(shape, lambda b: (0, 0))
    return pl.pallas_call(
        _enc_body,
        grid=(N_NODES // ROWB,),
        in_specs=[
            pl.BlockSpec((1, ROWB, HALF), lambda b: (0, b, 0)),
            pl.BlockSpec((1, ROWB, HALF), lambda b: (1, b, 0)),
            wspec((D, D)),
            wspec((D, H1)),
            wspec((D, H1)),
            wspec((H1, H1)),
        ],
        out_specs=pl.BlockSpec((ROWB, H1), lambda b: (b, 0)),
        out_shape=jax.ShapeDtypeStruct((N_NODES, H1), jnp.float32),
    )(agg2, agg2, W_rgc, W_u, W_i, Q)


def _dec_body(u_ref, v_ref, out_ref):
    z = lax.dot_general(u_ref[...], v_ref[...], (((1,), (1,)), ((), ())),
                        preferred_element_type=jnp.float32)
    out_ref[...] = 1.0 / (1.0 + jnp.exp(-z))


def _decode(uqi):
    return pl.pallas_call(
        _dec_body,
        grid=(NBLK_U,),
        in_specs=[
            pl.BlockSpec((ROWB, H1), lambda i: (i, 0)),
            pl.BlockSpec((N_USERS, H1), lambda i: (1, 0)),
        ],
        out_specs=pl.BlockSpec((ROWB, N_USERS), lambda i: (i, 0)),
        out_shape=jax.ShapeDtypeStruct((N_USERS, N_USERS), jnp.float32),
    )(uqi, uqi)


def kernel(x, edge_index, edge_norm, W_rgc, W_u, W_i, Q):
    xr = x.reshape(N_NODES * 2, HALF)
    src = edge_index[0]
    src2 = jnp.concatenate([src * 2, src * 2 + 1])
    norm16 = jnp.broadcast_to(edge_norm[:, None], (N_EDGES, LANES)).reshape(-1)
    agg2 = _sc_agg(xr, src2, edge_index[1], norm16)
    uqi = _encode(agg2, W_rgc, W_u, W_i, Q)
    out = _decode(uqi)
    return out.reshape(N_USERS * N_USERS, 1)


# speedup vs baseline: 1.8307x; 1.0709x over previous
import functools

import jax
import jax.numpy as jnp
from jax import lax
from jax.experimental import pallas as pl
from jax.experimental.pallas import tpu as pltpu
from jax.experimental.pallas import tpu_sc as plsc

N_USERS = 5000
N_NODES = 10000
D = 256
HALF = 128
H1 = 128
N_EDGES = 320000

NC = 2
NS = 16
LANES = 16

CHUNK = 128
NCHUNKS = 157
EDGES_PER_TILE = NCHUNKS * CHUNK
NE_PAD = NS * EDGES_PER_TILE
EREC = 3 * CHUNK
ROWS_A = 632
ROWS_B = N_NODES - 15 * ROWS_A


def _sc_agg_body(xr_hbm, edata_hbm, out_hbm,
                 agg_sh, ed_v, idx_v, dst_v, rows_v, sem_e, sem_g):
    c = lax.axis_index("c")
    s = lax.axis_index("s")

    def zero_row(r, _):
        for j in range(HALF // LANES):
            rows_v[0, r, pl.ds(j * LANES, LANES)] = jnp.zeros((LANES,), jnp.float32)
        return 0
    lax.fori_loop(0, CHUNK, zero_row, 0)
    r0 = pl.multiple_of(s * ROWS_A, 8)

    @pl.when(s < 15)
    def _():
        for k in range(4):
            pltpu.sync_copy(rows_v.at[0], agg_sh.at[pl.ds(r0 + k * CHUNK, CHUNK)])
        pltpu.sync_copy(rows_v.at[0, pl.ds(0, 120)], agg_sh.at[pl.ds(r0 + 512, 120)])

    @pl.when(s == 15)
    def _():
        for k in range(4):
            pltpu.sync_copy(rows_v.at[0], agg_sh.at[pl.ds(15 * ROWS_A + k * CHUNK, CHUNK)])
        pltpu.sync_copy(rows_v.at[0, pl.ds(0, 8)],
                        agg_sh.at[pl.ds(15 * ROWS_A + 512, 8)])
    plsc.subcore_barrier()

    def load_idx(b):
        for j in range(HALF // LANES):
            sl = pl.ds(j * LANES, LANES)
            idx_v[b, sl] = ed_v[b, sl].astype(jnp.int32) + c
            dst_v[b, sl] = ed_v[b, pl.ds(CHUNK + j * LANES, LANES)].astype(jnp.int32)

    base = s * NCHUNKS
    pltpu.sync_copy(edata_hbm.at[pl.ds(base * EREC, EREC)], ed_v.at[0, pl.ds(0, EREC)])
    load_idx(0)
    pltpu.async_copy(xr_hbm.at[idx_v.at[0]], rows_v.at[0], sem_g.at[0])

    def chunk_body(k, _):
        b = jnp.bitwise_and(k, 1)
        b2 = 1 - b
        not_last = k < NCHUNKS - 1

        @pl.when(not_last)
        def _():
            pltpu.async_copy(edata_hbm.at[pl.ds((base + k + 1) * EREC, EREC)],
                             ed_v.at[b2, pl.ds(0, EREC)], sem_e.at[b2])

        pltpu.make_async_copy(xr_hbm.at[idx_v.at[b]], rows_v.at[b],
                              sem_g.at[b]).wait()

        def group_body(g, _):
            nvg = ed_v[b, pl.ds(2 * CHUNK + g * LANES, LANES)]
            for ri in range(LANES):
                r = g * LANES + ri
                nv = nvg[ri]
                for j in range(HALF // LANES):
                    sl = pl.ds(j * LANES, LANES)
                    rows_v[b, r, sl] = rows_v[b, r, sl] * nv
            return 0
        lax.fori_loop(0, CHUNK // LANES, group_body, 0)

        @pl.when(not_last)
        def _():
            pltpu.make_async_copy(edata_hbm.at[pl.ds((base + k + 1) * EREC, EREC)],
                                  ed_v.at[b2, pl.ds(0, EREC)], sem_e.at[b2]).wait()
            load_idx(b2)
            pltpu.async_copy(xr_hbm.at[idx_v.at[b2]], rows_v.at[b2], sem_g.at[b2])

        pltpu.sync_copy(rows_v.at[b], agg_sh.at[dst_v.at[b]], add=True)
        return 0
    lax.fori_loop(0, NCHUNKS, chunk_body, 0)

    plsc.subcore_barrier()

    @pl.when(s < 15)
    def _():
        pltpu.sync_copy(agg_sh.at[pl.ds(r0, ROWS_A)], out_hbm.at[c, pl.ds(r0, ROWS_A)])

    @pl.when(s == 15)
    def _():
        pltpu.sync_copy(agg_sh.at[pl.ds(15 * ROWS_A, ROWS_B)],
                        out_hbm.at[c, pl.ds(15 * ROWS_A, ROWS_B)])


_sc_agg = functools.partial(
    pl.kernel,
    mesh=plsc.VectorSubcoreMesh(core_axis_name="c", subcore_axis_name="s"),
    out_type=jax.ShapeDtypeStruct((NC, N_NODES, HALF), jnp.float32),
    scratch_types=[
        pltpu.VMEM_SHARED((N_NODES, HALF), jnp.float32),
        pltpu.VMEM((2, EREC + LANES), jnp.float32),
        pltpu.VMEM((2, CHUNK), jnp.int32),
        pltpu.VMEM((2, CHUNK), jnp.int32),
        pltpu.VMEM((2, CHUNK, HALF), jnp.float32),
        pltpu.SemaphoreType.DMA((2,)),
        pltpu.SemaphoreType.DMA((2,)),
    ],
)(_sc_agg_body)


ROWB = 1000
NBLK_U = N_USERS // ROWB


def _enc_body(aL_ref, aR_ref, Wr_ref, Wu_ref, Wi_ref, Q_ref, out_ref):
    b = pl.program_id(0)
    aL = aL_ref[0]
    aR = aR_ref[0]
    Wr = Wr_ref[...]
    feats = jnp.maximum(
        jnp.dot(aL, Wr[:HALF], preferred_element_type=jnp.float32)
        + jnp.dot(aR, Wr[HALF:], preferred_element_type=jnp.float32), 0.0)
    is_user = b < NBLK_U
    W2 = jnp.where(is_user, Wu_ref[...], Wi_ref[...])
    h = jnp.maximum(jnp.dot(feats, W2, preferred_element_type=jnp.float32), 0.0)
    hq = jnp.dot(h, Q_ref[...], preferred_element_type=jnp.float32)
    out_ref[...] = jnp.where(is_user, hq, h)


def _encode(agg2, W_rgc, W_u, W_i, Q):
    wspec = lambda shape: pl.BlockSpec(shape, lambda b: (0, 0))
    return pl.pallas_call(
        _enc_body,
        grid=(N_NODES // ROWB,),
        in_specs=[
            pl.BlockSpec((1, ROWB, HALF), lambda b: (0, b, 0)),
            pl.BlockSpec((1, ROWB, HALF), lambda b: (1, b, 0)),
            wspec((D, D)),
            wspec((D, H1)),
            wspec((D, H1)),
            wspec((H1, H1)),
        ],
        out_specs=pl.BlockSpec((ROWB, H1), lambda b: (b, 0)),
        out_shape=jax.ShapeDtypeStruct((N_NODES, H1), jnp.float32),
    )(agg2, agg2, W_rgc, W_u, W_i, Q)


def _dec_body(u_ref, v_ref, out_ref):
    z = lax.dot_general(u_ref[...], v_ref[...], (((1,), (1,)), ((), ())),
                        preferred_element_type=jnp.float32)
    out_ref[...] = 1.0 / (1.0 + jnp.exp(-z))


def _decode(uqi):
    return pl.pallas_call(
        _dec_body,
        grid=(NBLK_U,),
        in_specs=[
            pl.BlockSpec((ROWB, H1), lambda i: (i, 0)),
            pl.BlockSpec((N_USERS, H1), lambda i: (1, 0)),
        ],
        out_specs=pl.BlockSpec((ROWB, N_USERS), lambda i: (i, 0)),
        out_shape=jax.ShapeDtypeStruct((N_USERS, N_USERS), jnp.float32),
    )(uqi, uqi)


def kernel(x, edge_index, edge_norm, W_rgc, W_u, W_i, Q):
    xr = x.reshape(N_NODES * 2, HALF)
    pad = NE_PAD - N_EDGES
    src2 = jnp.concatenate([edge_index[0] * 2, jnp.zeros((pad,), jnp.int32)])
    dstp = jnp.concatenate([edge_index[1],
                            jnp.arange(pad, dtype=jnp.int32) % N_NODES])
    normp = jnp.concatenate([edge_norm, jnp.zeros((pad,), jnp.float32)])
    edata = jnp.stack([
        src2.astype(jnp.float32).reshape(-1, CHUNK),
        dstp.astype(jnp.float32).reshape(-1, CHUNK),
        normp.reshape(-1, CHUNK),
    ], axis=1).reshape(-1)
    agg2 = _sc_agg(xr, edata)
    uqi = _encode(agg2, W_rgc, W_u, W_i, Q)
    out = _decode(uqi)
    return out.reshape(N_USERS * N_USERS, 1)


# speedup vs baseline: 2.7609x; 1.5081x over previous
import functools

import jax
import jax.numpy as jnp
from jax import lax
from jax.experimental import pallas as pl
from jax.experimental.pallas import tpu as pltpu
from jax.experimental.pallas import tpu_sc as plsc

N_USERS = 5000
N_NODES = 10000
D = 256
HALF = 128
H1 = 128
N_EDGES = 320000

NC = 2
NS = 16
LANES = 16

CHUNK = 64
NCHUNKS = 316
EDGES_PER_TILE = NCHUNKS * CHUNK
NE_PAD = NS * EDGES_PER_TILE
EREC = 3 * CHUNK
ROWS_A = 632
ROWS_B = N_NODES - 15 * ROWS_A


NBUF = 4
GDEPTH = 3


def _sc_agg_body(xr_hbm, edata_hbm, out_hbm, agg_sh,
                 ed0, ed1, ed2, ed3, ix0, ix1, ix2, ix3,
                 dv0, dv1, dv2, dv3, rw0, rw1, rw2, rw3, sem_e, sem_g):
    c = lax.axis_index("c")
    s = lax.axis_index("s")
    eds = [ed0, ed1, ed2, ed3]
    ixs = [ix0, ix1, ix2, ix3]
    dvs = [dv0, dv1, dv2, dv3]
    rws = [rw0, rw1, rw2, rw3]

    def zero_row(r, _):
        for j in range(HALF // LANES):
            rw0[r, pl.ds(j * LANES, LANES)] = jnp.zeros((LANES,), jnp.float32)
        return 0
    lax.fori_loop(0, CHUNK, zero_row, 0)
    r0 = pl.multiple_of(s * ROWS_A, 8)

    nfa, rema = divmod(ROWS_A, CHUNK)
    nfb, remb = divmod(ROWS_B, CHUNK)

    @pl.when(s < 15)
    def _():
        for k in range(nfa):
            pltpu.sync_copy(rw0, agg_sh.at[pl.ds(r0 + k * CHUNK, CHUNK)])
        if rema:
            pltpu.sync_copy(rw0.at[pl.ds(0, rema)],
                            agg_sh.at[pl.ds(r0 + nfa * CHUNK, rema)])

    @pl.when(s == 15)
    def _():
        for k in range(nfb):
            pltpu.sync_copy(rw0, agg_sh.at[pl.ds(15 * ROWS_A + k * CHUNK, CHUNK)])
        if remb:
            pltpu.sync_copy(rw0.at[pl.ds(0, remb)],
                            agg_sh.at[pl.ds(15 * ROWS_A + nfb * CHUNK, remb)])
    plsc.subcore_barrier()

    base = s * NCHUNKS

    def ed_dma(k, slot):
        return pltpu.make_async_copy(
            edata_hbm.at[pl.ds((base + k) * EREC, EREC)],
            eds[slot].at[pl.ds(0, EREC)], sem_e.at[slot])

    def load_idx(slot):
        for j in range(CHUNK // LANES):
            ixs[slot][pl.ds(j * LANES, LANES)] = (
                eds[slot][pl.ds(j * LANES, LANES)].astype(jnp.int32) + c)
            dvs[slot][pl.ds(j * LANES, LANES)] = (
                eds[slot][pl.ds(CHUNK + j * LANES, LANES)].astype(jnp.int32))

    for i in range(GDEPTH):
        ed_dma(i, i).start()
        ed_dma(i, i).wait()
        load_idx(i)
        pltpu.async_copy(xr_hbm.at[ixs[i]], rws[i], sem_g.at[i])
    ed_dma(GDEPTH, GDEPTH).start()

    def super_body(kk, _):
        for i in range(NBUF):
            k = kk * NBUF + i
            pltpu.make_async_copy(xr_hbm.at[ixs[i]], rws[i], sem_g.at[i]).wait()

            def group_body(g, _, i=i):
                nvg = eds[i][pl.ds(2 * CHUNK + g * LANES, LANES)]
                for ri in range(LANES):
                    r = g * LANES + ri
                    nv = nvg[ri]
                    for j in range(HALF // LANES):
                        sl = pl.ds(j * LANES, LANES)
                        rws[i][r, sl] = rws[i][r, sl] * nv
                return 0
            lax.fori_loop(0, CHUNK // LANES, group_body, 0)

            kn = k + GDEPTH
            jn = (i + GDEPTH) % NBUF

            @pl.when(kn < NCHUNKS)
            def _(kn=kn, jn=jn):
                ed_dma(kn, jn).wait()
                load_idx(jn)
                pltpu.async_copy(xr_hbm.at[ixs[jn]], rws[jn], sem_g.at[jn])

            @pl.when(kn + 1 < NCHUNKS)
            def _(kn=kn, i=i):
                ed_dma(kn + 1, i).start()

            pltpu.sync_copy(rws[i], agg_sh.at[dvs[i]], add=True)
        return 0
    lax.fori_loop(0, NCHUNKS // NBUF, super_body, 0)

    plsc.subcore_barrier()

    @pl.when(s < 15)
    def _():
        pltpu.sync_copy(agg_sh.at[pl.ds(r0, ROWS_A)], out_hbm.at[c, pl.ds(r0, ROWS_A)])

    @pl.when(s == 15)
    def _():
        pltpu.sync_copy(agg_sh.at[pl.ds(15 * ROWS_A, ROWS_B)],
                        out_hbm.at[c, pl.ds(15 * ROWS_A, ROWS_B)])


_sc_agg = functools.partial(
    pl.kernel,
    mesh=plsc.VectorSubcoreMesh(core_axis_name="c", subcore_axis_name="s"),
    out_type=jax.ShapeDtypeStruct((NC, N_NODES, HALF), jnp.float32),
    scratch_types=(
        [pltpu.VMEM_SHARED((N_NODES, HALF), jnp.float32)]
        + [pltpu.VMEM((EREC + LANES,), jnp.float32)] * NBUF
        + [pltpu.VMEM((CHUNK,), jnp.int32)] * NBUF
        + [pltpu.VMEM((CHUNK,), jnp.int32)] * NBUF
        + [pltpu.VMEM((CHUNK, HALF), jnp.float32)] * NBUF
        + [pltpu.SemaphoreType.DMA((NBUF,)),
           pltpu.SemaphoreType.DMA((NBUF,))]
    ),
)(_sc_agg_body)


ROWB = 1000
NBLK_U = N_USERS // ROWB


def _enc_body(aL_ref, aR_ref, Wr_ref, Wu_ref, Wi_ref, Q_ref, out_ref):
    b = pl.program_id(0)
    aL = aL_ref[0]
    aR = aR_ref[0]
    Wr = Wr_ref[...]
    feats = jnp.maximum(
        jnp.dot(aL, Wr[:HALF], preferred_element_type=jnp.float32)
        + jnp.dot(aR, Wr[HALF:], preferred_element_type=jnp.float32), 0.0)
    is_user = b < NBLK_U
    W2 = jnp.where(is_user, Wu_ref[...], Wi_ref[...])
    h = jnp.maximum(jnp.dot(feats, W2, preferred_element_type=jnp.float32), 0.0)
    hq = jnp.dot(h, Q_ref[...], preferred_element_type=jnp.float32)
    out_ref[...] = jnp.where(is_user, hq, h)


def _encode(agg2, W_rgc, W_u, W_i, Q):
    wspec = lambda shape: pl.BlockSpec(shape, lambda b: (0, 0))
    return pl.pallas_call(
        _enc_body,
        grid=(N_NODES // ROWB,),
        in_specs=[
            pl.BlockSpec((1, ROWB, HALF), lambda b: (0, b, 0)),
            pl.BlockSpec((1, ROWB, HALF), lambda b: (1, b, 0)),
            wspec((D, D)),
            wspec((D, H1)),
            wspec((D, H1)),
            wspec((H1, H1)),
        ],
        out_specs=pl.BlockSpec((ROWB, H1), lambda b: (b, 0)),
        out_shape=jax.ShapeDtypeStruct((N_NODES, H1), jnp.float32),
    )(agg2, agg2, W_rgc, W_u, W_i, Q)


def _dec_body(u_ref, v_ref, out_ref):
    z = lax.dot_general(u_ref[...], v_ref[...], (((1,), (1,)), ((), ())),
                        preferred_element_type=jnp.float32)
    out_ref[...] = 1.0 / (1.0 + jnp.exp(-z))


def _decode(uqi):
    return pl.pallas_call(
        _dec_body,
        grid=(NBLK_U,),
        in_specs=[
            pl.BlockSpec((ROWB, H1), lambda i: (i, 0)),
            pl.BlockSpec((N_USERS, H1), lambda i: (1, 0)),
        ],
        out_specs=pl.BlockSpec((ROWB, N_USERS), lambda i: (i, 0)),
        out_shape=jax.ShapeDtypeStruct((N_USERS, N_USERS), jnp.float32),
    )(uqi, uqi)


def kernel(x, edge_index, edge_norm, W_rgc, W_u, W_i, Q):
    xr = x.reshape(N_NODES * 2, HALF)
    pad = NE_PAD - N_EDGES
    src2 = jnp.concatenate([edge_index[0] * 2, jnp.zeros((pad,), jnp.int32)])
    dstp = jnp.concatenate([edge_index[1],
                            jnp.arange(pad, dtype=jnp.int32) % N_NODES])
    normp = jnp.concatenate([edge_norm, jnp.zeros((pad,), jnp.float32)])
    edata = jnp.stack([
        src2.astype(jnp.float32).reshape(-1, CHUNK),
        dstp.astype(jnp.float32).reshape(-1, CHUNK),
        normp.reshape(-1, CHUNK),
    ], axis=1).reshape(-1)
    agg2 = _sc_agg(xr, edata)
    uqi = _encode(agg2, W_rgc, W_u, W_i, Q)
    out = _decode(uqi)
    return out.reshape(N_USERS * N_USERS, 1)
